# Initial kernel scaffold; baseline (speedup 1.0000x reference)
import jax
import jax.numpy as jnp
from jax.experimental import pallas as pl


def kernel(node_features, latent_features, edge_features, edge_index, W1, b1, W2, b2, Wg, bg, Wl, bl, gamma, beta):
    raise NotImplementedError("write your pallas kernel here")


# speedup vs baseline: 4.1976x; 4.1976x over previous
import functools

import jax
import jax.numpy as jnp
from jax import lax
from jax.experimental import pallas as pl
from jax.experimental.pallas import tpu as pltpu
from jax.experimental.pallas import tpu_sc as plsc

_N = 10000
_E = 160000
_CS = 128
_CL = 64
_CZ = 64
_DEF = 107

_CH = 128
_NW = 32
_NCHUNK = _E // _CH

_NB = 2000
_BE = 2000


def _precompute_body(n_ref, l_ref, wns_ref, wls_ref, wnd_ref, wld_ref, b1_ref,
                     ps_ref, pd_ref):
    n = n_ref[...]
    lt = l_ref[...]
    ps_ref[...] = (
        jnp.dot(n, wns_ref[...], preferred_element_type=jnp.float32)
        + jnp.dot(lt, wls_ref[...], preferred_element_type=jnp.float32)
        + b1_ref[...]
    )
    pd_ref[...] = (
        jnp.dot(n, wnd_ref[...], preferred_element_type=jnp.float32)
        + jnp.dot(lt, wld_ref[...], preferred_element_type=jnp.float32)
    )


def _precompute(node, latent, wns, wls, wnd, wld, b1):
    grid = _N // _NB
    return pl.pallas_call(
        _precompute_body,
        grid=(grid,),
        in_specs=[
            pl.BlockSpec((_NB, _CS), lambda i: (i, 0)),
            pl.BlockSpec((_NB, _CL), lambda i: (i, 0)),
            pl.BlockSpec((_CS, _CZ), lambda i: (0, 0)),
            pl.BlockSpec((_CL, _CZ), lambda i: (0, 0)),
            pl.BlockSpec((_CS, _CZ), lambda i: (0, 0)),
            pl.BlockSpec((_CL, _CZ), lambda i: (0, 0)),
            pl.BlockSpec((1, _CZ), lambda i: (0, 0)),
        ],
        out_specs=[
            pl.BlockSpec((_NB, _CZ), lambda i: (i, 0)),
            pl.BlockSpec((_NB, _CZ), lambda i: (i, 0)),
        ],
        out_shape=[
            jax.ShapeDtypeStruct((_N, _CZ), jnp.float32),
            jax.ShapeDtypeStruct((_N, _CZ), jnp.float32),
        ],
    )(node, latent, wns, wls, wnd, wld, b1.reshape(1, _CZ))


def _make_gather():
    mesh = plsc.VectorSubcoreMesh(core_axis_name="c", subcore_axis_name="s")

    @functools.partial(
        pl.kernel,
        mesh=mesh,
        out_type=(
            jax.ShapeDtypeStruct((_E, _CZ), jnp.float32),
            jax.ShapeDtypeStruct((_E, _CZ), jnp.float32),
        ),
        scratch_types=[
            pltpu.VMEM((_CH,), jnp.int32),
            pltpu.VMEM((_CH,), jnp.int32),
            pltpu.VMEM((_CH, _CZ), jnp.float32),
            pltpu.VMEM((_CH, _CZ), jnp.float32),
            pltpu.SemaphoreType.DMA,
            pltpu.SemaphoreType.DMA,
        ],
        compiler_params=pltpu.CompilerParams(use_tc_tiling_on_sc=False),
    )
    def gather_kernel(psrc_hbm, pdst_hbm, src_hbm, dst_hbm, gs_hbm, gd_hbm,
                      idx_s, idx_d, rows_s, rows_d, sem_s, sem_d):
        wid = lax.axis_index("s") * 2 + lax.axis_index("c")
        nbase = _NCHUNK // _NW
        extra = _NCHUNK % _NW
        n_me = nbase + jnp.where(wid < extra, 1, 0)
        start = wid * nbase + jnp.minimum(wid, extra)

        def body(t, carry):
            off = (start + t) * _CH
            pltpu.sync_copy(src_hbm.at[pl.ds(off, _CH)], idx_s)
            pltpu.sync_copy(dst_hbm.at[pl.ds(off, _CH)], idx_d)
            cp_s = pltpu.async_copy(psrc_hbm.at[idx_s], rows_s, sem_s)
            cp_d = pltpu.async_copy(pdst_hbm.at[idx_d], rows_d, sem_d)
            cp_s.wait()
            cp_d.wait()
            pltpu.sync_copy(rows_s, gs_hbm.at[pl.ds(off, _CH)])
            pltpu.sync_copy(rows_d, gd_hbm.at[pl.ds(off, _CH)])
            return carry

        lax.fori_loop(0, n_me, body, 0)

    return gather_kernel


_gather_cache = []


def _gather(ps, pd, src, dst):
    if not _gather_cache:
        _gather_cache.append(_make_gather())
    return _gather_cache[0](ps, pd, src, dst)


def _mlp_body(ef_ref, gs_ref, gd_ref, w1_ref, w2_ref, b2_ref, wg_ref, bg_ref,
              wl_ref, bl_ref, gamma_ref, beta_ref, out_ref):
    x = jnp.dot(ef_ref[...], w1_ref[...], preferred_element_type=jnp.float32)
    x = jnp.maximum(x + gs_ref[...] + gd_ref[...], 0.0)
    x = jnp.dot(x, w2_ref[...], preferred_element_type=jnp.float32) + b2_ref[...]
    x = jnp.maximum(x, 0.0)
    gate = jnp.dot(x, wg_ref[...], preferred_element_type=jnp.float32) + bg_ref[...]
    lin = jnp.dot(x, wl_ref[...], preferred_element_type=jnp.float32) + bl_ref[...]
    y = lin * jax.nn.sigmoid(gate)
    mean = jnp.mean(y, axis=-1, keepdims=True)
    yc = y - mean
    var = jnp.mean(yc * yc, axis=-1, keepdims=True)
    out_ref[...] = yc * lax.rsqrt(var + 1e-5) * gamma_ref[...] + beta_ref[...]


def _mlp(ef, gs, gd, w1ef, w2, b2, wg, bg, wl, bl, gamma, beta):
    grid = _E // _BE
    return pl.pallas_call(
        _mlp_body,
        grid=(grid,),
        in_specs=[
            pl.BlockSpec((_BE, _DEF), lambda i: (i, 0)),
            pl.BlockSpec((_BE, _CZ), lambda i: (i, 0)),
            pl.BlockSpec((_BE, _CZ), lambda i: (i, 0)),
            pl.BlockSpec((_DEF, _CZ), lambda i: (0, 0)),
            pl.BlockSpec((_CZ, _CZ), lambda i: (0, 0)),
            pl.BlockSpec((1, _CZ), lambda i: (0, 0)),
            pl.BlockSpec((_CZ, _CZ), lambda i: (0, 0)),
            pl.BlockSpec((1, _CZ), lambda i: (0, 0)),
            pl.BlockSpec((_CZ, _CZ), lambda i: (0, 0)),
            pl.BlockSpec((1, _CZ), lambda i: (0, 0)),
            pl.BlockSpec((1, _CZ), lambda i: (0, 0)),
            pl.BlockSpec((1, _CZ), lambda i: (0, 0)),
        ],
        out_specs=pl.BlockSpec((_BE, _CZ), lambda i: (i, 0)),
        out_shape=jax.ShapeDtypeStruct((_E, _CZ), jnp.float32),
    )(ef, gs, gd, w1ef, w2, b2.reshape(1, _CZ), wg, bg.reshape(1, _CZ),
      wl, bl.reshape(1, _CZ), gamma.reshape(1, _CZ), beta.reshape(1, _CZ))


def kernel(node_features, latent_features, edge_features, edge_index,
           W1, b1, W2, b2, Wg, bg, Wl, bl, gamma, beta):
    w1ef = W1[:_DEF]
    wns = W1[_DEF:_DEF + _CS]
    wnd = W1[_DEF + _CS:_DEF + 2 * _CS]
    wls = W1[_DEF + 2 * _CS:_DEF + 2 * _CS + _CL]
    wld = W1[_DEF + 2 * _CS + _CL:]
    src = edge_index[0].astype(jnp.int32)
    dst = edge_index[1].astype(jnp.int32)

    ps, pd = _precompute(node_features, latent_features, wns, wls, wnd, wld, b1)
    gs, gd = _gather(ps, pd, src, dst)
    return _mlp(edge_features, gs, gd, w1ef, W2, b2, Wg, bg, Wl, bl,
                gamma, beta)
